# Initial kernel scaffold; baseline (speedup 1.0000x reference)
import jax
import jax.numpy as jnp
from jax.experimental import pallas as pl


def kernel(node_features, edge_index, edge_attr, W1a, b1a, W1b, b1b, W2a, b2a, W2b, b2b):
    raise NotImplementedError("write your pallas kernel here")


# speedup vs baseline: 3.2406x; 3.2406x over previous
import functools

import jax
import jax.numpy as jnp
from jax import lax
from jax.experimental import pallas as pl
from jax.experimental.pallas import tpu as pltpu
from jax.experimental.pallas import tpu_sc as plsc

NC = 2
NS = 16
LANES = 128
G = 8


def _sc_agg_fn(n_nodes, d, chunks, rows_per_tile):
    r_total = NS * rows_per_tile
    n_groups = chunks // G
    mesh = plsc.VectorSubcoreMesh(
        core_axis_name="c", subcore_axis_name="s", num_cores=NC, num_subcores=NS)

    @functools.partial(
        pl.kernel,
        mesh=mesh,
        out_type=jax.ShapeDtypeStruct((NC, r_total, d), jnp.float32),
        scratch_types=[
            pltpu.VMEM((2, G, LANES), jnp.int32),
            pltpu.VMEM((2, G, LANES), jnp.int32),
            pltpu.VMEM((LANES, d), jnp.float32),
            pltpu.VMEM((LANES, d), jnp.float32),
            pltpu.VMEM_SHARED((r_total, d), jnp.float32),
            pltpu.SemaphoreType.DMA,
            pltpu.SemaphoreType.DMA,
            pltpu.SemaphoreType.DMA,
        ],
    )
    def sc_agg(h_hbm, src_hbm, dst_hbm, z_hbm, out_hbm,
               src_v, dst_v, rows_a, rows_b, agg_sh, sem_a, sem_b, sem_i):
        c = lax.axis_index("c")
        s = lax.axis_index("s")
        wid = s * NC + c
        row0 = s * rows_per_tile

        pltpu.sync_copy(src_hbm.at[wid, pl.ds(0, G)], src_v.at[0])
        pltpu.sync_copy(dst_hbm.at[wid, pl.ds(0, G)], dst_v.at[0])
        pltpu.sync_copy(z_hbm, rows_a)

        def zbody(k, carry):
            pltpu.sync_copy(rows_a, agg_sh.at[pl.ds(row0 + k * LANES, LANES)])
            return carry
        lax.fori_loop(0, rows_per_tile // LANES, zbody, 0)

        pltpu.async_copy(h_hbm.at[src_v.at[0, 0]], rows_a, sem_a)
        plsc.subcore_barrier()

        def group_body(g, carry):
            gb = g % 2
            nb = (g + 1) % 2

            @pl.when(g + 1 < n_groups)
            def _():
                pltpu.async_copy(
                    src_hbm.at[wid, pl.ds((g + 1) * G, G)], src_v.at[nb], sem_i)
                pltpu.async_copy(
                    dst_hbm.at[wid, pl.ds((g + 1) * G, G)], dst_v.at[nb], sem_i)

            for jj in range(G):
                cur, csem = (rows_a, sem_a) if jj % 2 == 0 else (rows_b, sem_b)
                nxt, nsem = (rows_b, sem_b) if jj % 2 == 0 else (rows_a, sem_a)
                pltpu.make_async_copy(
                    h_hbm.at[src_v.at[gb, jj]], cur, csem).wait()
                if jj + 1 < G:
                    pltpu.async_copy(h_hbm.at[src_v.at[gb, jj + 1]], nxt, nsem)
                else:
                    @pl.when(g + 1 < n_groups)
                    def _():
                        pltpu.make_async_copy(
                            src_hbm.at[wid, pl.ds((g + 1) * G, G)],
                            src_v.at[nb], sem_i).wait()
                        pltpu.make_async_copy(
                            dst_hbm.at[wid, pl.ds((g + 1) * G, G)],
                            dst_v.at[nb], sem_i).wait()
                        pltpu.async_copy(h_hbm.at[src_v.at[nb, 0]], nxt, nsem)
                pltpu.sync_copy(cur, agg_sh.at[dst_v.at[gb, jj]], add=True)
            return carry
        lax.fori_loop(0, n_groups, group_body, 0)

        plsc.subcore_barrier()

        def wbody(k, carry):
            r = row0 + k * LANES
            pltpu.sync_copy(agg_sh.at[pl.ds(r, LANES)], rows_a)
            pltpu.sync_copy(rows_a, out_hbm.at[c, pl.ds(r, LANES)])
            return carry
        lax.fori_loop(0, rows_per_tile // LANES, wbody, 0)

    return sc_agg


def _mlp_body(h_ref, agg_ref, wa_ref, ba_ref, wb_ref, bb_ref, out_ref):
    z = h_ref[...] + agg_ref[0] + agg_ref[1]
    a = jnp.dot(z, wa_ref[...], preferred_element_type=jnp.float32)
    a = jnp.maximum(a + ba_ref[...], 0.0)
    o = jnp.dot(a, wb_ref[...], preferred_element_type=jnp.float32)
    out_ref[...] = jnp.maximum(o + bb_ref[...], 0.0)


def _mlp(h, agg, wat, ba, wbt, bb, block_rows):
    n, d = h.shape
    grid = n // block_rows
    return pl.pallas_call(
        _mlp_body,
        grid=(grid,),
        in_specs=[
            pl.BlockSpec((block_rows, d), lambda i: (i, 0)),
            pl.BlockSpec((NC, block_rows, d), lambda i: (0, i, 0)),
            pl.BlockSpec((d, d), lambda i: (0, 0)),
            pl.BlockSpec((1, d), lambda i: (0, 0)),
            pl.BlockSpec((d, d), lambda i: (0, 0)),
            pl.BlockSpec((1, d), lambda i: (0, 0)),
        ],
        out_specs=pl.BlockSpec((block_rows, d), lambda i: (i, 0)),
        out_shape=jax.ShapeDtypeStruct((n, d), jnp.float32),
    )(h, agg, wat, ba, wbt, bb)


def kernel(node_features, edge_index, edge_attr,
           W1a, b1a, W1b, b1b, W2a, b2a, W2b, b2b):
    n, d = node_features.shape
    e = edge_index.shape[1]

    per = NC * NS * LANES
    chunks = -(-(-(-e // per)) // G) * G
    e_pad = chunks * per
    src = jnp.concatenate(
        [edge_index[0], jnp.zeros((e_pad - e,), jnp.int32)]).reshape(
            NC * NS, chunks, LANES)
    dst = jnp.concatenate(
        [edge_index[1], jnp.full((e_pad - e,), n, jnp.int32)]).reshape(
            NC * NS, chunks, LANES)

    rows_per_tile = -(-(n + 1) // (NS * LANES)) * LANES
    zeros = jnp.zeros((LANES, d), jnp.float32)

    sc_agg = _sc_agg_fn(n, d, chunks, rows_per_tile)

    block_rows = 1000 if n % 1000 == 0 else 8
    ba, bb1 = b1a.reshape(1, d), b1b.reshape(1, d)
    agg1 = sc_agg(node_features, src, dst, zeros)
    h1 = _mlp(node_features, agg1, W1a.T, ba, W1b.T, bb1, block_rows)
    agg2 = sc_agg(h1, src, dst, zeros)
    h2 = _mlp(h1, agg2, W2a.T, b2a.reshape(1, d), W2b.T, b2b.reshape(1, d),
              block_rows)
    return h2


# speedup vs baseline: 3.2442x; 1.0011x over previous
import functools

import jax
import jax.numpy as jnp
from jax import lax
from jax.experimental import pallas as pl
from jax.experimental.pallas import tpu as pltpu
from jax.experimental.pallas import tpu_sc as plsc

NC = 2
NS = 16
LANES = 128
G = 8


def _sc_agg_fn(n_nodes, d, chunks, rows_per_tile):
    r_total = NS * rows_per_tile
    n_groups = chunks // G
    mesh = plsc.VectorSubcoreMesh(
        core_axis_name="c", subcore_axis_name="s", num_cores=NC, num_subcores=NS)

    @functools.partial(
        pl.kernel,
        mesh=mesh,
        out_type=jax.ShapeDtypeStruct((NC, r_total, d), jnp.float32),
        scratch_types=[
            pltpu.VMEM((2, G, LANES), jnp.int32),
            pltpu.VMEM((2, G, LANES), jnp.int32),
            pltpu.VMEM((LANES, d), jnp.float32),
            pltpu.VMEM((LANES, d), jnp.float32),
            pltpu.VMEM_SHARED((r_total, d), jnp.float32),
            pltpu.SemaphoreType.DMA,
            pltpu.SemaphoreType.DMA,
            pltpu.SemaphoreType.DMA,
        ],
    )
    def sc_agg(h_hbm, src_hbm, dst_hbm, z_hbm, out_hbm,
               src_v, dst_v, rows_a, rows_b, agg_sh, sem_a, sem_b, sem_i):
        c = lax.axis_index("c")
        s = lax.axis_index("s")
        wid = s * NC + c
        row0 = s * rows_per_tile

        pltpu.sync_copy(src_hbm.at[wid, pl.ds(0, G)], src_v.at[0])
        pltpu.sync_copy(dst_hbm.at[wid, pl.ds(0, G)], dst_v.at[0])
        pltpu.sync_copy(z_hbm, rows_a)

        def zbody(k, carry):
            pltpu.sync_copy(rows_a, agg_sh.at[pl.ds(row0 + k * LANES, LANES)])
            return carry
        lax.fori_loop(0, rows_per_tile // LANES, zbody, 0)

        pltpu.async_copy(h_hbm.at[src_v.at[0, 0]], rows_a, sem_a)
        plsc.subcore_barrier()

        def group_body(g, carry):
            gb = g % 2
            nb = (g + 1) % 2

            @pl.when(g + 1 < n_groups)
            def _():
                pltpu.async_copy(
                    src_hbm.at[wid, pl.ds((g + 1) * G, G)], src_v.at[nb], sem_i)
                pltpu.async_copy(
                    dst_hbm.at[wid, pl.ds((g + 1) * G, G)], dst_v.at[nb], sem_i)

            for jj in range(G):
                cur, csem = (rows_a, sem_a) if jj % 2 == 0 else (rows_b, sem_b)
                nxt, nsem = (rows_b, sem_b) if jj % 2 == 0 else (rows_a, sem_a)
                pltpu.make_async_copy(
                    h_hbm.at[src_v.at[gb, jj]], cur, csem).wait()
                if jj + 1 < G:
                    pltpu.async_copy(h_hbm.at[src_v.at[gb, jj + 1]], nxt, nsem)
                else:
                    @pl.when(g + 1 < n_groups)
                    def _():
                        pltpu.make_async_copy(
                            src_hbm.at[wid, pl.ds((g + 1) * G, G)],
                            src_v.at[nb], sem_i).wait()
                        pltpu.make_async_copy(
                            dst_hbm.at[wid, pl.ds((g + 1) * G, G)],
                            dst_v.at[nb], sem_i).wait()
                        pltpu.async_copy(h_hbm.at[src_v.at[nb, 0]], nxt, nsem)
                pltpu.sync_copy(cur, agg_sh.at[dst_v.at[gb, jj]], add=True)
            return carry
        lax.fori_loop(0, n_groups, group_body, 0)

        plsc.subcore_barrier()

        def wbody(k, carry):
            r = row0 + k * LANES
            pltpu.sync_copy(agg_sh.at[pl.ds(r, LANES)], rows_a)
            pltpu.sync_copy(rows_a, out_hbm.at[c, pl.ds(r, LANES)])
            return carry
        lax.fori_loop(0, rows_per_tile // LANES, wbody, 0)

    return sc_agg


def _mlp_body(h_ref, agg_ref, wa_ref, ba_ref, wb_ref, bb_ref, out_ref):
    z = h_ref[...] + agg_ref[0] + agg_ref[1]
    a = jnp.dot(z, wa_ref[...], preferred_element_type=jnp.float32)
    a = jnp.maximum(a + ba_ref[...], 0.0)
    o = jnp.dot(a, wb_ref[...], preferred_element_type=jnp.float32)
    out_ref[...] = jnp.maximum(o + bb_ref[...], 0.0)


def _mlp(h, agg, wat, ba, wbt, bb, block_rows):
    n, d = h.shape
    grid = n // block_rows
    return pl.pallas_call(
        _mlp_body,
        grid=(grid,),
        in_specs=[
            pl.BlockSpec((block_rows, d), lambda i: (i, 0)),
            pl.BlockSpec((NC, block_rows, d), lambda i: (0, i, 0)),
            pl.BlockSpec((d, d), lambda i: (0, 0)),
            pl.BlockSpec((1, d), lambda i: (0, 0)),
            pl.BlockSpec((d, d), lambda i: (0, 0)),
            pl.BlockSpec((1, d), lambda i: (0, 0)),
        ],
        out_specs=pl.BlockSpec((block_rows, d), lambda i: (i, 0)),
        out_shape=jax.ShapeDtypeStruct((n, d), jnp.float32),
    )(h, agg, wat, ba, wbt, bb)


def kernel(node_features, edge_index, edge_attr,
           W1a, b1a, W1b, b1b, W2a, b2a, W2b, b2b):
    n, d = node_features.shape
    e = edge_index.shape[1]

    per = NC * NS * LANES
    chunks = -(-(-(-e // per)) // G) * G
    e_pad = chunks * per
    rows_per_tile = -(-(n + 1) // (NS * LANES)) * LANES
    r_total = NS * rows_per_tile

    pad = e_pad - e
    src = jnp.concatenate(
        [edge_index[0], jnp.zeros((pad,), jnp.int32)]).reshape(
            NC * NS, chunks, LANES)
    dst_pad = n + jnp.arange(pad, dtype=jnp.int32) % max(r_total - n, 1)
    dst = jnp.concatenate([edge_index[1], dst_pad]).reshape(
        NC * NS, chunks, LANES)
    zeros = jnp.zeros((LANES, d), jnp.float32)

    sc_agg = _sc_agg_fn(n, d, chunks, rows_per_tile)

    block_rows = 1000 if n % 1000 == 0 else 8
    ba, bb1 = b1a.reshape(1, d), b1b.reshape(1, d)
    agg1 = sc_agg(node_features, src, dst, zeros)
    h1 = _mlp(node_features, agg1, W1a.T, ba, W1b.T, bb1, block_rows)
    agg2 = sc_agg(h1, src, dst, zeros)
    h2 = _mlp(h1, agg2, W2a.T, b2a.reshape(1, d), W2b.T, b2b.reshape(1, d),
              block_rows)
    return h2
